# baseline (speedup 1.0000x reference)
import functools

import jax
import jax.numpy as jnp
from jax import lax
from jax.experimental import pallas as pl
from jax.experimental.pallas import tpu as pltpu
from jax.experimental.pallas import tpu_sc as plsc

_N, _M, _C = 20000, 100, 80
_R, _L = 160, 128
_NP = _R * _L
_B = 256
_BIG = 2**30


def _iou_block(P1, P2, P3, P4, PA, gx1, gy1, gx2, gy2, ga):
    x1 = jnp.maximum(P1, gx1)
    y1 = jnp.maximum(P2, gy1)
    x2 = jnp.minimum(P3, gx2)
    y2 = jnp.minimum(P4, gy2)
    inter = jnp.maximum(x2 - x1, 0.0) * jnp.maximum(y2 - y1, 0.0)
    union = (PA + ga) - inter
    return inter / jnp.maximum(union, 1e-9)


_S = 512
_NCH = _NP // _S


def _match_body(p_ref, gt_ref, ps_ref, gtv_ref,
                mp_ref, val_ref,
                pa_ref, idx_ref, best_ref, arg_ref, pen_ref,
                acc_ref, aidx_ref):
    P1, P2, P3, P4 = p_ref[0], p_ref[1], p_ref[2], p_ref[3]
    pa_ref[...] = (P3 - P1) * (P4 - P2)
    ridx = lax.broadcasted_iota(jnp.int32, (_R, _L), 0)
    cidx = lax.broadcasted_iota(jnp.int32, (_R, _L), 1)
    idx_ref[...] = ridx * _L + cidx
    lane = lax.broadcasted_iota(jnp.int32, (1, _L), 1)
    best_ref[...] = jnp.full((1, _L), -jnp.inf, jnp.float32)
    arg_ref[...] = jnp.zeros((1, _L), jnp.int32)
    pen_ref[...] = jnp.zeros((_R, _L), jnp.float32)
    mp_ref[...] = jnp.zeros((2, _L), jnp.int32)
    val_ref[...] = jnp.zeros((2, _L), jnp.float32)

    def col_scan(j):
        gx1 = gt_ref[j, 0]
        gy1 = gt_ref[j, 1]
        gx2 = gt_ref[j, 2]
        gy2 = gt_ref[j, 3]
        ga = (gx2 - gx1) * (gy2 - gy1)
        iou = _iou_block(P1, P2, P3, P4, pa_ref[...],
                         gx1, gy1, gx2, gy2, ga) + pen_ref[...]
        m = jnp.max(iou)
        f = jnp.min(jnp.where(iou == m, idx_ref[...], _BIG))
        return m, f

    gx1v = gtv_ref[0]
    gy1v = gtv_ref[1]
    gx2v = gtv_ref[2]
    gy2v = gtv_ref[3]
    gav = (gx2v - gx1v) * (gy2v - gy1v)
    acc_ref[...] = jnp.full((_S, _L), -jnp.inf, jnp.float32)
    aidx_ref[...] = jnp.zeros((_S, _L), jnp.int32)
    subi = lax.broadcasted_iota(jnp.int32, (_S, 1), 0)

    def chunk_c(c, carry):
        base = c * _S
        cx1 = ps_ref[pl.ds(base, _S), 0:1]
        cy1 = ps_ref[pl.ds(base, _S), 1:2]
        cx2 = ps_ref[pl.ds(base, _S), 2:3]
        cy2 = ps_ref[pl.ds(base, _S), 3:4]
        cpa = (cx2 - cx1) * (cy2 - cy1)
        iou = _iou_block(cx1, cy1, cx2, cy2, cpa,
                         gx1v, gy1v, gx2v, gy2v, gav)
        upd = iou > acc_ref[...]
        acc_ref[...] = jnp.where(upd, iou, acc_ref[...])
        aidx_ref[...] = jnp.where(upd, base + subi, aidx_ref[...])
        return carry

    lax.fori_loop(0, _NCH, chunk_c, 0, unroll=2)

    accv = acc_ref[...]
    aidxv = aidx_ref[...]
    h = _S
    while h > 1:
        h //= 2
        a_v, b_v = accv[0:h, :], accv[h:2 * h, :]
        a_i, b_i = aidxv[0:h, :], aidxv[h:2 * h, :]
        take = jnp.logical_or(b_v > a_v,
                              jnp.logical_and(b_v == a_v, b_i < a_i))
        accv = jnp.where(take, b_v, a_v)
        aidxv = jnp.where(take, b_i, a_i)
    best_ref[...] = accv
    arg_ref[...] = aidxv


    def pen_write(f):
        row_f = f // _L
        lane_f = f % _L
        prow = pen_ref[pl.ds(row_f, 1), :]
        pen_ref[pl.ds(row_f, 1), :] = jnp.where(lane == lane_f,
                                                -jnp.inf, prow)

    def rescan_one(l):
        def rcond(c):
            return c

        def rbody(c):
            m2, f2 = col_scan(l)
            taken = jnp.max(jnp.where(
                jnp.logical_and(mp_ref[0:1, :] == f2,
                                val_ref[0:1, :] > 0.5), 1, 0)) > 0
            hit = jnp.logical_and(taken, m2 > -jnp.inf)

            @pl.when(hit)
            def _():
                pen_write(f2)

            @pl.when(jnp.logical_not(hit))
            def _():
                onlane = lane == l
                best_ref[...] = jnp.where(onlane, m2, best_ref[...])
                arg_ref[...] = jnp.where(onlane, f2, arg_ref[...])

            return hit

        lax.while_loop(rcond, rbody, jnp.bool_(True))

    def round_body(c):
        B = best_ref[...]
        A = arg_ref[...]
        actf = jnp.where(B >= 0.5, 1.0, 0.0)
        Bm = jnp.where(B >= 0.5, B, -jnp.inf)
        F = (_L, _L)
        Acol = jnp.broadcast_to(A, F)
        Bcol = jnp.broadcast_to(Bm, F)
        ActC = jnp.broadcast_to(actf, F)
        AT = jnp.transpose(Acol)
        BT = jnp.transpose(Bcol)
        ActT = jnp.transpose(ActC)
        same = jnp.logical_and(Acol == AT,
                               jnp.logical_and(ActC > 0.5, ActT > 0.5))
        Mx = jnp.where(same, Bcol, -jnp.inf)
        grpmax = jnp.max(Mx, axis=1, keepdims=True)
        kio = lax.broadcasted_iota(jnp.int32, F, 1)
        winlane = jnp.min(jnp.where(Mx == grpmax, kio, _BIG),
                          axis=1, keepdims=True)
        sub = lax.broadcasted_iota(jnp.int32, (_L, 1), 0)
        actrow = ActT[:, 0:1] > 0.5
        Brow = BT[:, 0:1]
        win_j = jnp.logical_and(winlane == sub, actrow)
        loser_j = jnp.logical_and(actrow, jnp.logical_not(win_j))
        maxloser = jnp.max(jnp.where(loser_j, Brow, -jnp.inf))
        mxv = jnp.max(Bm)
        safe_j = jnp.logical_and(win_j,
                                 jnp.logical_or(Brow > maxloser,
                                                Brow == mxv))
        safef = jnp.where(safe_j, 1.0, 0.0)
        safeL = jnp.transpose(jnp.broadcast_to(safef, F))[0:1, :]
        onsafe = safeL > 0.5
        mp_ref[0:1, :] = jnp.where(onsafe, A, mp_ref[0:1, :])
        val_ref[0:1, :] = jnp.where(onsafe, 1.0, val_ref[0:1, :])
        best_ref[...] = jnp.where(onsafe, -jnp.inf, B)
        SafeC = jnp.broadcast_to(safeL, F)
        takenrow = jnp.max(jnp.where(jnp.logical_and(same, SafeC > 0.5),
                                     1.0, 0.0), axis=1, keepdims=True)
        needf = jnp.where(jnp.logical_and(loser_j, takenrow > 0.5),
                          1.0, 0.0)
        needL = jnp.transpose(jnp.broadcast_to(needf, F))[0:1, :]

        def lane_cond(nd):
            return jnp.max(nd) > 0.5

        def lane_body(nd):
            l = jnp.min(jnp.where(nd > 0.5, lane, _BIG))
            rescan_one(l)
            return jnp.where(lane == l, 0.0, nd)

        lax.while_loop(lane_cond, lane_body, needL)
        return jnp.max(best_ref[...]) >= 0.5

    cont0 = jnp.max(best_ref[...]) >= 0.5
    lax.while_loop(lambda c: c, round_body, cont0)


def _loss_body(x_ref, g_ref, vm_ref, out_ref):
    vm = vm_ref[...]
    cnt = jnp.sum(vm)
    X = x_ref[...]
    G = g_ref[...]
    lane = lax.broadcasted_iota(jnp.int32, (_B, _L), 1)

    Xc = jnp.where(lane < _C, X, -jnp.inf)
    m = jnp.max(Xc, axis=1, keepdims=True)
    s = jnp.sum(jnp.where(lane < _C, jnp.exp(X - m), 0.0),
                axis=1, keepdims=True)
    lse = jnp.log(s) + m
    clsf = jnp.sum(jnp.where(lane == _C + 4, G, 0.0), axis=1, keepdims=True)
    xc = jnp.sum(jnp.where(lane.astype(jnp.float32) == clsf, X, 0.0),
                 axis=1, keepdims=True)
    ce_sum = jnp.sum((lse - xc) * vm)

    d = X - G
    ad = jnp.abs(d)
    sl1 = jnp.where(ad < 1.0, 0.5 * d * d, ad - 0.5)
    boxmask = jnp.logical_and(lane >= _C, lane < _C + 4)
    box_sum = jnp.sum(jnp.where(boxmask, sl1, 0.0) * vm)

    lane1 = lax.broadcasted_iota(jnp.int32, (1, _L), 1)
    cden = jnp.maximum(cnt, 1.0)
    out_ref[...] = jnp.where(lane1 == 0, ce_sum / cden,
                             jnp.where(lane1 == 1, box_sum / (cden * 4.0),
                                       0.0))


def _gather_rows(table, mp2):
    info = plsc.get_sparse_core_info()
    nw = info.num_cores * info.num_subcores
    bpw = _B // nw
    nsub = info.num_subcores
    mesh = plsc.VectorSubcoreMesh(core_axis_name="c", subcore_axis_name="s")

    @functools.partial(
        pl.kernel,
        out_type=jax.ShapeDtypeStruct((_B, _L), jnp.float32),
        mesh=mesh,
        scratch_types=[
            pltpu.VMEM((bpw,), jnp.int32),
            pltpu.VMEM((bpw, _L), jnp.float32),
            pltpu.SemaphoreType.DMA,
        ],
        compiler_params=pltpu.CompilerParams(use_tc_tiling_on_sc=True),
    )
    def sc_gather(tab_hbm, mp_hbm, out_hbm, mp_v, rows_v, sem):
        wid = lax.axis_index("s") * info.num_cores + lax.axis_index("c")
        base = wid * bpw
        row = wid // nsub
        off = (wid % nsub) * bpw
        pltpu.sync_copy(mp_hbm.at[row, pl.ds(off, bpw)], mp_v)
        pltpu.async_copy(tab_hbm.at[mp_v], rows_v, sem).wait()
        pltpu.sync_copy(rows_v, out_hbm.at[pl.ds(base, bpw)])

    return sc_gather(table, mp2)


def kernel(cls_scores, pred_boxes, gt_boxes, gt_classes):
    pb = pred_boxes.astype(jnp.float32)
    pred_pad = jnp.pad(pb, ((0, _NP - _N), (0, 0)))
    P = pred_pad.T.reshape(4, _R, _L)
    gt_b = gt_boxes.astype(jnp.float32)
    gtv = jnp.pad(gt_b.T, ((0, 0), (0, _L - _M))).reshape(4, 1, _L)
    table = jnp.concatenate(
        [cls_scores.astype(jnp.float32), pb,
         jnp.zeros((_N, _L - _C - 4), jnp.float32)], axis=1)
    gt_big = jnp.concatenate(
        [jnp.zeros((_M, _C), jnp.float32), gt_b,
         gt_classes.astype(jnp.float32)[:, None],
         jnp.zeros((_M, _L - _C - 5), jnp.float32)], axis=1)
    gt_big = jnp.pad(gt_big, ((0, _B - _M), (0, 0)))

    mp, valid = pl.pallas_call(
        _match_body,
        out_shape=[
            jax.ShapeDtypeStruct((2, _L), jnp.int32),
            jax.ShapeDtypeStruct((2, _L), jnp.float32),
        ],
        in_specs=[
            pl.BlockSpec(memory_space=pltpu.VMEM),
            pl.BlockSpec(memory_space=pltpu.SMEM),
            pl.BlockSpec(memory_space=pltpu.VMEM),
            pl.BlockSpec(memory_space=pltpu.VMEM),
        ],
        out_specs=[pl.BlockSpec(memory_space=pltpu.VMEM)] * 2,
        scratch_shapes=[
            pltpu.VMEM((_R, _L), jnp.float32),
            pltpu.VMEM((_R, _L), jnp.int32),
            pltpu.VMEM((1, _L), jnp.float32),
            pltpu.VMEM((1, _L), jnp.int32),
            pltpu.VMEM((_R, _L), jnp.float32),
            pltpu.VMEM((_S, _L), jnp.float32),
            pltpu.VMEM((_S, _L), jnp.int32),
        ],
    )(P, gt_b, pred_pad, gtv)

    rows = _gather_rows(table, mp)

    out = pl.pallas_call(
        _loss_body,
        out_shape=jax.ShapeDtypeStruct((1, _L), jnp.float32),
        in_specs=[pl.BlockSpec(memory_space=pltpu.VMEM)] * 3,
        out_specs=pl.BlockSpec(memory_space=pltpu.VMEM),
    )(rows, gt_big, valid.reshape(_B, 1))

    return out[0, 0], out[0, 1]


# speedup vs baseline: 1.2604x; 1.2604x over previous
import functools

import jax
import jax.numpy as jnp
from jax import lax
from jax.experimental import pallas as pl
from jax.experimental.pallas import tpu as pltpu
from jax.experimental.pallas import tpu_sc as plsc

_N, _M, _C = 20000, 100, 80
_R, _L = 160, 128
_NP = _R * _L
_B = 256
_BIG = 2**30


def _iou_block(P1, P2, P3, P4, PA, gx1, gy1, gx2, gy2, ga):
    x1 = jnp.maximum(P1, gx1)
    y1 = jnp.maximum(P2, gy1)
    x2 = jnp.minimum(P3, gx2)
    y2 = jnp.minimum(P4, gy2)
    inter = jnp.maximum(x2 - x1, 0.0) * jnp.maximum(y2 - y1, 0.0)
    union = (PA + ga) - inter
    return inter / jnp.maximum(union, 1e-9)


def _match_body(p_ref, gt_ref,
                mp_ref, val_ref,
                pa_ref, idx_ref, best_ref, arg_ref, pen_ref):
    P1, P2, P3, P4 = p_ref[0], p_ref[1], p_ref[2], p_ref[3]
    pa_ref[...] = (P3 - P1) * (P4 - P2)
    ridx = lax.broadcasted_iota(jnp.int32, (_R, _L), 0)
    cidx = lax.broadcasted_iota(jnp.int32, (_R, _L), 1)
    idx_ref[...] = ridx * _L + cidx
    lane = lax.broadcasted_iota(jnp.int32, (1, _L), 1)
    best_ref[...] = jnp.full((1, _L), -jnp.inf, jnp.float32)
    arg_ref[...] = jnp.zeros((1, _L), jnp.int32)
    pen_ref[...] = jnp.zeros((_R, _L), jnp.float32)
    mp_ref[...] = jnp.zeros((2, _L), jnp.int32)
    val_ref[...] = jnp.zeros((2, _L), jnp.float32)

    def col_scan(j):
        gx1 = gt_ref[j, 0]
        gy1 = gt_ref[j, 1]
        gx2 = gt_ref[j, 2]
        gy2 = gt_ref[j, 3]
        ga = (gx2 - gx1) * (gy2 - gy1)
        iou = _iou_block(P1, P2, P3, P4, pa_ref[...],
                         gx1, gy1, gx2, gy2, ga) + pen_ref[...]
        m = jnp.max(iou)
        f = jnp.min(jnp.where(iou == m, idx_ref[...], _BIG))
        return m, f

    def init_j(j, carry):
        m, f = col_scan(j)
        onlane = lane == j
        best_ref[...] = jnp.where(onlane, m, best_ref[...])
        arg_ref[...] = jnp.where(onlane, f, arg_ref[...])
        return carry

    lax.fori_loop(0, _M, init_j, 0, unroll=10)


    def pen_write(f):
        row_f = f // _L
        lane_f = f % _L
        prow = pen_ref[pl.ds(row_f, 1), :]
        pen_ref[pl.ds(row_f, 1), :] = jnp.where(lane == lane_f,
                                                -jnp.inf, prow)

    def rescan_one(l):
        def rcond(c):
            return c

        def rbody(c):
            m2, f2 = col_scan(l)
            taken = jnp.max(jnp.where(
                jnp.logical_and(mp_ref[0:1, :] == f2,
                                val_ref[0:1, :] > 0.5), 1, 0)) > 0
            hit = jnp.logical_and(taken, m2 > -jnp.inf)

            @pl.when(hit)
            def _():
                pen_write(f2)

            @pl.when(jnp.logical_not(hit))
            def _():
                onlane = lane == l
                best_ref[...] = jnp.where(onlane, m2, best_ref[...])
                arg_ref[...] = jnp.where(onlane, f2, arg_ref[...])

            return hit

        lax.while_loop(rcond, rbody, jnp.bool_(True))

    def round_body(c):
        B = best_ref[...]
        A = arg_ref[...]
        actf = jnp.where(B >= 0.5, 1.0, 0.0)
        Bm = jnp.where(B >= 0.5, B, -jnp.inf)
        F = (_L, _L)
        Acol = jnp.broadcast_to(A, F)
        Bcol = jnp.broadcast_to(Bm, F)
        ActC = jnp.broadcast_to(actf, F)
        AT = jnp.transpose(Acol)
        BT = jnp.transpose(Bcol)
        ActT = jnp.transpose(ActC)
        same = jnp.logical_and(Acol == AT,
                               jnp.logical_and(ActC > 0.5, ActT > 0.5))
        Mx = jnp.where(same, Bcol, -jnp.inf)
        grpmax = jnp.max(Mx, axis=1, keepdims=True)
        kio = lax.broadcasted_iota(jnp.int32, F, 1)
        winlane = jnp.min(jnp.where(Mx == grpmax, kio, _BIG),
                          axis=1, keepdims=True)
        sub = lax.broadcasted_iota(jnp.int32, (_L, 1), 0)
        actrow = ActT[:, 0:1] > 0.5
        Brow = BT[:, 0:1]
        win_j = jnp.logical_and(winlane == sub, actrow)
        loser_j = jnp.logical_and(actrow, jnp.logical_not(win_j))
        maxloser = jnp.max(jnp.where(loser_j, Brow, -jnp.inf))
        mxv = jnp.max(Bm)
        safe_j = jnp.logical_and(win_j,
                                 jnp.logical_or(Brow > maxloser,
                                                Brow == mxv))
        safef = jnp.where(safe_j, 1.0, 0.0)
        safeL = jnp.transpose(jnp.broadcast_to(safef, F))[0:1, :]
        onsafe = safeL > 0.5
        mp_ref[0:1, :] = jnp.where(onsafe, A, mp_ref[0:1, :])
        val_ref[0:1, :] = jnp.where(onsafe, 1.0, val_ref[0:1, :])
        best_ref[...] = jnp.where(onsafe, -jnp.inf, B)
        SafeC = jnp.broadcast_to(safeL, F)
        takenrow = jnp.max(jnp.where(jnp.logical_and(same, SafeC > 0.5),
                                     1.0, 0.0), axis=1, keepdims=True)
        needf = jnp.where(jnp.logical_and(loser_j, takenrow > 0.5),
                          1.0, 0.0)
        needL = jnp.transpose(jnp.broadcast_to(needf, F))[0:1, :]

        def lane_cond(nd):
            return jnp.max(nd) > 0.5

        def lane_body(nd):
            l = jnp.min(jnp.where(nd > 0.5, lane, _BIG))
            rescan_one(l)
            return jnp.where(lane == l, 0.0, nd)

        lax.while_loop(lane_cond, lane_body, needL)
        return jnp.max(best_ref[...]) >= 0.5

    cont0 = jnp.max(best_ref[...]) >= 0.5
    lax.while_loop(lambda c: c, round_body, cont0)


def _loss_body(x_ref, g_ref, vm_ref, out_ref):
    vm = vm_ref[...]
    cnt = jnp.sum(vm)
    X = x_ref[...]
    G = g_ref[...]
    lane = lax.broadcasted_iota(jnp.int32, (_B, _L), 1)

    Xc = jnp.where(lane < _C, X, -jnp.inf)
    m = jnp.max(Xc, axis=1, keepdims=True)
    s = jnp.sum(jnp.where(lane < _C, jnp.exp(X - m), 0.0),
                axis=1, keepdims=True)
    lse = jnp.log(s) + m
    clsf = jnp.sum(jnp.where(lane == _C + 4, G, 0.0), axis=1, keepdims=True)
    xc = jnp.sum(jnp.where(lane.astype(jnp.float32) == clsf, X, 0.0),
                 axis=1, keepdims=True)
    ce_sum = jnp.sum((lse - xc) * vm)

    d = X - G
    ad = jnp.abs(d)
    sl1 = jnp.where(ad < 1.0, 0.5 * d * d, ad - 0.5)
    boxmask = jnp.logical_and(lane >= _C, lane < _C + 4)
    box_sum = jnp.sum(jnp.where(boxmask, sl1, 0.0) * vm)

    lane1 = lax.broadcasted_iota(jnp.int32, (1, _L), 1)
    cden = jnp.maximum(cnt, 1.0)
    out_ref[...] = jnp.where(lane1 == 0, ce_sum / cden,
                             jnp.where(lane1 == 1, box_sum / (cden * 4.0),
                                       0.0))


def _gather_rows(table, mp2):
    info = plsc.get_sparse_core_info()
    nw = info.num_cores * info.num_subcores
    bpw = _B // nw
    nsub = info.num_subcores
    mesh = plsc.VectorSubcoreMesh(core_axis_name="c", subcore_axis_name="s")

    @functools.partial(
        pl.kernel,
        out_type=jax.ShapeDtypeStruct((_B, _L), jnp.float32),
        mesh=mesh,
        scratch_types=[
            pltpu.VMEM((bpw,), jnp.int32),
            pltpu.VMEM((bpw, _L), jnp.float32),
            pltpu.SemaphoreType.DMA,
        ],
        compiler_params=pltpu.CompilerParams(use_tc_tiling_on_sc=True),
    )
    def sc_gather(tab_hbm, mp_hbm, out_hbm, mp_v, rows_v, sem):
        wid = lax.axis_index("s") * info.num_cores + lax.axis_index("c")
        base = wid * bpw
        row = wid // nsub
        off = (wid % nsub) * bpw
        pltpu.sync_copy(mp_hbm.at[row, pl.ds(off, bpw)], mp_v)
        pltpu.async_copy(tab_hbm.at[mp_v], rows_v, sem).wait()
        pltpu.sync_copy(rows_v, out_hbm.at[pl.ds(base, bpw)])

    return sc_gather(table, mp2)


def kernel(cls_scores, pred_boxes, gt_boxes, gt_classes):
    pb = pred_boxes.astype(jnp.float32)
    pred_pad = jnp.pad(pb, ((0, _NP - _N), (0, 0)))
    P = pred_pad.T.reshape(4, _R, _L)
    gt_b = gt_boxes.astype(jnp.float32)
    table = jnp.concatenate(
        [cls_scores.astype(jnp.float32), pb,
         jnp.zeros((_N, _L - _C - 4), jnp.float32)], axis=1)
    gt_big = jnp.concatenate(
        [jnp.zeros((_M, _C), jnp.float32), gt_b,
         gt_classes.astype(jnp.float32)[:, None],
         jnp.zeros((_M, _L - _C - 5), jnp.float32)], axis=1)
    gt_big = jnp.pad(gt_big, ((0, _B - _M), (0, 0)))

    mp, valid = pl.pallas_call(
        _match_body,
        out_shape=[
            jax.ShapeDtypeStruct((2, _L), jnp.int32),
            jax.ShapeDtypeStruct((2, _L), jnp.float32),
        ],
        in_specs=[
            pl.BlockSpec(memory_space=pltpu.VMEM),
            pl.BlockSpec(memory_space=pltpu.SMEM),
        ],
        out_specs=[pl.BlockSpec(memory_space=pltpu.VMEM)] * 2,
        scratch_shapes=[
            pltpu.VMEM((_R, _L), jnp.float32),
            pltpu.VMEM((_R, _L), jnp.int32),
            pltpu.VMEM((1, _L), jnp.float32),
            pltpu.VMEM((1, _L), jnp.int32),
            pltpu.VMEM((_R, _L), jnp.float32),
        ],
    )(P, gt_b)

    rows = _gather_rows(table, mp)

    out = pl.pallas_call(
        _loss_body,
        out_shape=jax.ShapeDtypeStruct((1, _L), jnp.float32),
        in_specs=[pl.BlockSpec(memory_space=pltpu.VMEM)] * 3,
        out_specs=pl.BlockSpec(memory_space=pltpu.VMEM),
    )(rows, gt_big, valid.reshape(_B, 1))

    return out[0, 0], out[0, 1]
